# baseline (speedup 1.0000x reference)
import jax
import jax.numpy as jnp
from jax import lax
from jax.experimental import pallas as pl
from jax.experimental.pallas import tpu as pltpu
from jax.experimental.pallas import tpu_sc as plsc

_NUM_CORES = 2
_NUM_SUBCORES = 16
_NW = _NUM_CORES * _NUM_SUBCORES

_V = 50
_B = 1024 * _V
_D = 1024
_BPW = _B // _NW
_C = 40
_NCHUNK = _BPW // _C
_NBUF = 3


def _rep_body(table_ref, rep_ref):
    rep_ref[0] = table_ref[...]


def _replicate(table):
    rep3 = pl.pallas_call(
        _rep_body,
        grid=(_NW,),
        in_specs=[pl.BlockSpec((_V, _D), lambda i: (0, 0))],
        out_specs=pl.BlockSpec((1, _V, _D), lambda i: (i, 0, 0)),
        out_shape=jax.ShapeDtypeStruct((_NW, _V, _D), jnp.float32),
    )(table)
    return rep3.reshape(_NW * _V, _D)


def _body(
    idx_hbm, rep_hbm, out_hbm, idx_v, buf_v, gsem0, gsem1, gsem2, ssem0, ssem1, ssem2
):
    sid = lax.axis_index("s")
    wid = sid * _NUM_CORES + lax.axis_index("c")
    base = wid * _BPW

    gsems = [gsem0, gsem1, gsem2]
    ssems = [ssem0, ssem1, ssem2]
    pltpu.sync_copy(idx_hbm.at[pl.ds(base, _BPW)], idx_v)

    rep_off = wid * _V

    def rebase(i, carry):
        sl = pl.ds(i * 16, 16)
        idx_v[sl] = idx_v[sl] + rep_off
        return carry

    lax.fori_loop(0, _BPW // 16, rebase, 0)

    gd = [None] * _NCHUNK
    sd = [None] * _NCHUNK

    def start_gather(i):
        b = i % _NBUF
        gd[i] = pltpu.async_copy(
            rep_hbm.at[idx_v.at[pl.ds(i * _C, _C)]], buf_v.at[b], gsems[b]
        )

    def start_scatter(i):
        b = i % _NBUF
        sd[i] = pltpu.async_copy(
            buf_v.at[b], out_hbm.at[pl.ds(base + i * _C, _C)], ssems[b]
        )

    for i in range(_NBUF - 1):
        start_gather(i)
    for i in range(_NCHUNK):
        if i + _NBUF - 1 < _NCHUNK:
            if i - 1 >= 0:
                sd[i - 1].wait()
            start_gather(i + _NBUF - 1)
        gd[i].wait()
        start_scatter(i)
    sd[_NCHUNK - 2].wait()
    sd[_NCHUNK - 1].wait()


@jax.jit
def _lookup(indices_flat, table):
    rep = _replicate(table)
    mesh = plsc.VectorSubcoreMesh(core_axis_name="c", subcore_axis_name="s")
    f = pl.kernel(
        _body,
        out_type=jax.ShapeDtypeStruct((_B, _D), jnp.float32),
        mesh=mesh,
        scratch_types=[
            pltpu.VMEM((_BPW,), jnp.int32),
            pltpu.VMEM((_NBUF, _C, _D), jnp.float32),
            pltpu.SemaphoreType.DMA,
            pltpu.SemaphoreType.DMA,
            pltpu.SemaphoreType.DMA,
            pltpu.SemaphoreType.DMA,
            pltpu.SemaphoreType.DMA,
            pltpu.SemaphoreType.DMA,
        ],
    )
    return f(indices_flat, rep)


def kernel(indices, embedding_weight):
    b, t = indices.shape
    flat = indices.reshape(-1).astype(jnp.int32)
    out = _lookup(flat, embedding_weight)
    return out.reshape(b, t, _D)


# speedup vs baseline: 1.0278x; 1.0278x over previous
import jax
import jax.numpy as jnp
from jax import lax
from jax.experimental import pallas as pl
from jax.experimental.pallas import tpu as pltpu
from jax.experimental.pallas import tpu_sc as plsc

_NUM_CORES = 2
_NUM_SUBCORES = 16
_NW = _NUM_CORES * _NUM_SUBCORES

_V = 50
_VPAD = 64
_B = 1024 * _V
_D = 1024
_BPW = _B // _NW
_C = 40
_NCHUNK = _BPW // _C
_NBUF = 3


def _body(
    idx_hbm, table_hbm, out_hbm, idx_v, buf_v, gsem0, gsem1, gsem2, ssem0, ssem1, ssem2
):
    sid = lax.axis_index("s")
    wid = sid * _NUM_CORES + lax.axis_index("c")
    base = wid * _BPW

    gsems = [gsem0, gsem1, gsem2]
    ssems = [ssem0, ssem1, ssem2]
    pltpu.sync_copy(idx_hbm.at[pl.ds(base, _BPW)], idx_v)

    rep_off = wid * _VPAD

    def rebase(i, carry):
        sl = pl.ds(i * 16, 16)
        idx_v[sl] = idx_v[sl] + rep_off
        return carry

    lax.fori_loop(0, _BPW // 16, rebase, 0)

    gd = [None] * _NCHUNK
    sd = [None] * _NCHUNK

    def start_gather(i):
        b = i % _NBUF
        gd[i] = pltpu.async_copy(
            table_hbm.at[idx_v.at[pl.ds(i * _C, _C)]], buf_v.at[b], gsems[b]
        )

    def start_scatter(i):
        b = i % _NBUF
        sd[i] = pltpu.async_copy(
            buf_v.at[b], out_hbm.at[pl.ds(base + i * _C, _C)], ssems[b]
        )

    for i in range(_NBUF - 1):
        start_gather(i)
    for i in range(_NCHUNK):
        if i + _NBUF - 1 < _NCHUNK:
            if i - 1 >= 0:
                sd[i - 1].wait()
            start_gather(i + _NBUF - 1)
        gd[i].wait()
        start_scatter(i)
    sd[_NCHUNK - 2].wait()
    sd[_NCHUNK - 1].wait()


@jax.jit
def _lookup(indices_flat, table_rep):
    mesh = plsc.VectorSubcoreMesh(core_axis_name="c", subcore_axis_name="s")
    f = pl.kernel(
        _body,
        out_type=jax.ShapeDtypeStruct((_B, _D), jnp.float32),
        mesh=mesh,
        scratch_types=[
            pltpu.VMEM((_BPW,), jnp.int32),
            pltpu.VMEM((_NBUF, _C, _D), jnp.float32),
            pltpu.SemaphoreType.DMA,
            pltpu.SemaphoreType.DMA,
            pltpu.SemaphoreType.DMA,
            pltpu.SemaphoreType.DMA,
            pltpu.SemaphoreType.DMA,
            pltpu.SemaphoreType.DMA,
        ],
    )
    return f(indices_flat, table_rep)


def kernel(indices, embedding_weight):
    b, t = indices.shape
    flat = indices.reshape(-1).astype(jnp.int32)
    rep = jnp.zeros((_NW, _VPAD, _D), embedding_weight.dtype)
    rep = rep.at[:, : _V, :].set(embedding_weight[None])
    out = _lookup(flat, rep.reshape(_NW * _VPAD, _D))
    return out.reshape(b, t, _D)
